# speedup vs baseline: 1.0127x; 1.0127x over previous
import jax
import jax.numpy as jnp
from jax.experimental import pallas as pl
from jax.experimental.pallas import tpu as pltpu

VOCAB = 1000
EMBED_DIM = 16
B = 4096
L = 20
BB = 128


def _tc_body(ids_ref, embed_ref, W_ref, b_ref, out_ref):
    ids = ids_ref[...]
    vocab_iota = jax.lax.broadcasted_iota(jnp.int32, (BB, VOCAB), 1)
    counts = jnp.zeros((BB, VOCAB), jnp.float32)
    for l in range(L):
        counts = counts + (ids[:, l][:, None] == vocab_iota).astype(jnp.float32)
    pooled = jnp.dot(counts, embed_ref[...], preferred_element_type=jnp.float32)
    logits = jnp.dot(pooled, W_ref[...], preferred_element_type=jnp.float32)
    logits = logits * (1.0 / L) + b_ref[...]
    out_ref[...] = jnp.broadcast_to(logits[:, None, :], (BB, L, VOCAB))


def kernel(input_ids, embed, W, b):
    b2 = b.reshape(1, VOCAB)
    out = pl.pallas_call(
        _tc_body,
        grid=(B // BB,),
        in_specs=[
            pl.BlockSpec((BB, L), lambda i: (i, 0)),
            pl.BlockSpec((VOCAB, EMBED_DIM), lambda i: (0, 0)),
            pl.BlockSpec((EMBED_DIM, VOCAB), lambda i: (0, 0)),
            pl.BlockSpec((1, VOCAB), lambda i: (0, 0)),
        ],
        out_specs=pl.BlockSpec((BB, L, VOCAB), lambda i: (i, 0, 0)),
        out_shape=jax.ShapeDtypeStruct((B, L, VOCAB), jnp.float32),
    )(input_ids, embed, W, b2)
    return out
